# baseline (speedup 1.0000x reference)
import functools

import jax
import jax.numpy as jnp
from jax import lax
from jax.experimental import pallas as pl
from jax.experimental.pallas import tpu as pltpu
from jax.experimental.pallas import tpu_sc as plsc

VOCAB = 1000000
EMBED_DIM = 32
BATCH = 4096
SEQ = 200

NUM_CORES = 2
NUM_SUBCORES = 16
NUM_WORKERS = NUM_CORES * NUM_SUBCORES

N = BATCH * SEQ
CHUNK = 512
JOBS_PER_SEQ = BATCH // CHUNK
NUM_JOBS = SEQ * JOBS_PER_SEQ
JOBS_PER_W = NUM_JOBS // NUM_WORKERS
PAIRS = JOBS_PER_W // 2
LANE_TILES = CHUNK // 128
SUB_TILES = EMBED_DIM // 8
T5 = CHUNK * EMBED_DIM
RUN = T5 // SUB_TILES


def _body(idx_hbm, table_hbm, out_hbm,
          idx_all, rows0, rows1, t50, t51, gsem0, gsem1, ssem0, ssem1):
    wid = lax.axis_index("s") * NUM_CORES + lax.axis_index("c")
    job_base = wid * JOBS_PER_W
    iota16 = lax.iota(jnp.int32, 16)
    p0 = (iota16 // 8) * 4096 + (iota16 % 8) * 128
    p1 = p0 + 2 * 4096

    pltpu.sync_copy(idx_hbm.at[pl.ds(wid * JOBS_PER_W, JOBS_PER_W)], idx_all)

    def store_job(t5f, job_id, sem):
        s = job_id // JOBS_PER_SEQ
        c0 = (job_id % JOBS_PER_SEQ) * LANE_TILES * 1024
        for i in range(SUB_TILES):
            pltpu.async_copy(t5f.at[pl.ds(i * RUN, RUN)],
                             out_hbm.at[s, i, pl.ds(c0, RUN)], sem)

    def drain_store(t5f, sem):
        for i in range(SUB_TILES):
            pltpu.make_async_copy(t5f.at[pl.ds(i * RUN, RUN)],
                                  out_hbm.at[0, i, pl.ds(0, RUN)], sem).wait()

    def transpose(rows, t5f):
        @plsc.parallel_loop(0, CHUNK, unroll=8)
        def _(b):
            base = (b // 128) * 1024 + (b % 128)
            bb = jnp.full((16,), base, jnp.int32)
            plsc.store_scatter(t5f, [p0 + bb], rows[b, 0:16])
            plsc.store_scatter(t5f, [p1 + bb], rows[b, 16:32])

    pltpu.async_copy(table_hbm.at[idx_all.at[0]], rows0, gsem0)

    def pair(t, carry):
        ja = 2 * t
        jb = 2 * t + 1

        gb = pltpu.async_copy(table_hbm.at[idx_all.at[jb]], rows1, gsem1)

        pltpu.make_async_copy(table_hbm.at[idx_all.at[ja]], rows0, gsem0).wait()
        @pl.when(t > 0)
        def _():
            drain_store(t50, ssem0)
        transpose(rows0, t50)
        store_job(t50, job_base + ja, ssem0)

        nxt = jnp.minimum(2 * t + 2, JOBS_PER_W - 2)
        pltpu.async_copy(table_hbm.at[idx_all.at[nxt]], rows0, gsem0)

        gb.wait()
        @pl.when(t > 0)
        def _():
            drain_store(t51, ssem1)
        transpose(rows1, t51)
        store_job(t51, job_base + jb, ssem1)
        return carry

    lax.fori_loop(0, PAIRS, pair, 0)

    pltpu.make_async_copy(table_hbm.at[idx_all.at[JOBS_PER_W - 2]],
                          rows0, gsem0).wait()
    drain_store(t50, ssem0)
    drain_store(t51, ssem1)


VCHUNK = 1024
FULL_CHUNKS = 999424 // VCHUNK
TAIL512 = 999424
TAIL64 = 999936
TROWS = VOCAB * EMBED_DIM // 128


def _tbody(tab_t, patch128, tabr, vin, vin512, vin64, tout):
    wid = lax.axis_index("s") * NUM_CORES + lax.axis_index("c")
    iota16 = lax.iota(jnp.int32, 16)
    row_pat = iota16 // 4
    col_pat = (iota16 % 4) * 32

    def do_chunk(v0, vbuf, width):
        v0 = pl.multiple_of(v0, 128)
        pltpu.sync_copy(tab_t.at[:, pl.ds(v0, width)], vbuf)

        @plsc.parallel_loop(0, EMBED_DIM * (width // 16), unroll=8)
        def _(run):
            d = run // (width // 16)
            t0 = (run % (width // 16)) * 16
            rows = jnp.full((16,), t0 // 4, jnp.int32) + row_pat
            cols = jnp.full((16,), d, jnp.int32) + col_pat
            plsc.store_scatter(tout, [rows, cols], vbuf[d, pl.ds(t0, 16)])
        pltpu.sync_copy(tout.at[pl.ds(0, width // 4)],
                        tabr.at[pl.ds(pl.multiple_of(v0 // 4, 8), width // 4)])

    def chunk_loop(k, carry):
        c = wid + NUM_WORKERS * k

        @pl.when(c < FULL_CHUNKS)
        def _():
            do_chunk(c * VCHUNK, vin, VCHUNK)
        return carry

    lax.fori_loop(0, (FULL_CHUNKS + NUM_WORKERS - 1) // NUM_WORKERS,
                  chunk_loop, 0)

    @pl.when(wid == 30)
    def _():
        do_chunk(TAIL512, vin512, 512)

    @pl.when(wid == 31)
    def _():
        pltpu.sync_copy(patch128, vin64)
        pltpu.sync_copy(vin64, tabr.at[pl.ds(TAIL64 * EMBED_DIM // 128, 16)])


@jax.jit
def _transpose_table(tab_t, patch128):
    mesh = plsc.VectorSubcoreMesh(core_axis_name="c", subcore_axis_name="s")
    f = functools.partial(
        pl.kernel,
        mesh=mesh,
        out_type=jax.ShapeDtypeStruct((TROWS, 128), jnp.float32),
        scratch_types=[
            pltpu.VMEM((EMBED_DIM, VCHUNK), jnp.float32),
            pltpu.VMEM((EMBED_DIM, 512), jnp.float32),
            pltpu.VMEM((16, 128), jnp.float32),
            pltpu.VMEM((VCHUNK // 4, 128), jnp.float32),
        ],
        compiler_params=pltpu.CompilerParams(
            use_tc_tiling_on_sc=True, needs_layout_passes=False,
            disable_bounds_checks=True),
    )(_tbody)
    return f(tab_t, patch128)


@jax.jit
def _gather(idx_lin, table):
    mesh = plsc.VectorSubcoreMesh(core_axis_name="c", subcore_axis_name="s")
    f = functools.partial(
        pl.kernel,
        mesh=mesh,
        out_type=jax.ShapeDtypeStruct(
            (SEQ, SUB_TILES, (BATCH // 128) * 1024), jnp.float32),
        scratch_types=[
            pltpu.VMEM((JOBS_PER_W, CHUNK), jnp.int32),
            pltpu.VMEM((CHUNK, EMBED_DIM), jnp.float32),
            pltpu.VMEM((CHUNK, EMBED_DIM), jnp.float32),
            pltpu.VMEM((T5,), jnp.float32),
            pltpu.VMEM((T5,), jnp.float32),
            pltpu.SemaphoreType.DMA,
            pltpu.SemaphoreType.DMA,
            pltpu.SemaphoreType.DMA,
            pltpu.SemaphoreType.DMA,
        ],
        compiler_params=pltpu.CompilerParams(
            use_tc_tiling_on_sc=False, needs_layout_passes=False,
            disable_bounds_checks=True),
    )(_body)
    return f(idx_lin, table)


def kernel(sentence_tokens, embedding_table):
    idx_lin = sentence_tokens.T.reshape(NUM_JOBS, CHUNK).astype(jnp.int32)
    patch128 = embedding_table[TAIL64:, :].reshape(16, 128)
    tabr = _transpose_table(embedding_table.T, patch128)
    out3 = _gather(idx_lin, tabr.reshape(VOCAB, EMBED_DIM))
    out5 = out3.reshape(SEQ, SUB_TILES, BATCH // 128, 8, 128)
    res = out5.transpose(0, 1, 3, 2, 4).reshape(SEQ, EMBED_DIM, BATCH)
    return res.transpose(2, 0, 1)


# speedup vs baseline: 1.0554x; 1.0554x over previous
import functools

import jax
import jax.numpy as jnp
from jax import lax
from jax.experimental import pallas as pl
from jax.experimental.pallas import tpu as pltpu
from jax.experimental.pallas import tpu_sc as plsc

VOCAB = 1000000
EMBED_DIM = 32
BATCH = 4096
SEQ = 200

NUM_CORES = 2
NUM_SUBCORES = 16
NUM_WORKERS = NUM_CORES * NUM_SUBCORES

N = BATCH * SEQ
CHUNK = 512
JOBS_PER_SEQ = BATCH // CHUNK
NUM_JOBS = SEQ * JOBS_PER_SEQ
JOBS_PER_W = NUM_JOBS // NUM_WORKERS
PAIRS = JOBS_PER_W // 2
LANE_TILES = CHUNK // 128
SUB_TILES = EMBED_DIM // 8
T5 = CHUNK * EMBED_DIM
RUN = T5 // SUB_TILES


def _body(idx_hbm, table_hbm, out_hbm,
          idx_all, rows0, rows1, t50, t51, gsem0, gsem1, ssem0, ssem1):
    wid = lax.axis_index("s") * NUM_CORES + lax.axis_index("c")
    job_base = wid * JOBS_PER_W
    iota16 = lax.iota(jnp.int32, 16)
    p0 = (iota16 // 8) * 4096 + (iota16 % 8) * 128
    p1 = p0 + 2 * 4096

    pltpu.sync_copy(idx_hbm.at[pl.ds(wid * JOBS_PER_W, JOBS_PER_W)], idx_all)

    def store_job(t5f, job_id, sem):
        s = job_id // JOBS_PER_SEQ
        c0 = (job_id % JOBS_PER_SEQ) * LANE_TILES * 1024
        for i in range(SUB_TILES):
            pltpu.async_copy(t5f.at[pl.ds(i * RUN, RUN)],
                             out_hbm.at[s, i, pl.ds(c0, RUN)], sem)

    def drain_store(t5f, sem):
        for i in range(SUB_TILES):
            pltpu.make_async_copy(t5f.at[pl.ds(i * RUN, RUN)],
                                  out_hbm.at[0, i, pl.ds(0, RUN)], sem).wait()

    def transpose(rows, t5f):
        @plsc.parallel_loop(0, CHUNK, unroll=8)
        def _(b):
            base = (b // 128) * 1024 + (b % 128)
            bb = jnp.full((16,), base, jnp.int32)
            plsc.store_scatter(t5f, [p0 + bb], rows[b, 0:16])
            plsc.store_scatter(t5f, [p1 + bb], rows[b, 16:32])

    pltpu.async_copy(table_hbm.at[idx_all.at[0]], rows0, gsem0)

    def pair(t, carry):
        ja = 2 * t
        jb = 2 * t + 1

        gb = pltpu.async_copy(table_hbm.at[idx_all.at[jb]], rows1, gsem1)

        pltpu.make_async_copy(table_hbm.at[idx_all.at[ja]], rows0, gsem0).wait()
        @pl.when(t > 0)
        def _():
            drain_store(t50, ssem0)
        transpose(rows0, t50)
        store_job(t50, job_base + ja, ssem0)

        nxt = jnp.minimum(2 * t + 2, JOBS_PER_W - 2)
        pltpu.async_copy(table_hbm.at[idx_all.at[nxt]], rows0, gsem0)

        gb.wait()
        @pl.when(t > 0)
        def _():
            drain_store(t51, ssem1)
        transpose(rows1, t51)
        store_job(t51, job_base + jb, ssem1)
        return carry

    lax.fori_loop(0, PAIRS, pair, 0)

    pltpu.make_async_copy(table_hbm.at[idx_all.at[JOBS_PER_W - 2]],
                          rows0, gsem0).wait()
    drain_store(t50, ssem0)
    drain_store(t51, ssem1)


VCHUNK = 1024
FULL_CHUNKS = 999424 // VCHUNK
TAIL512 = 999424
TAIL64 = 999936
TROWS = VOCAB * EMBED_DIM // 128


def _tbody(tab_t, patch_flat, tabr, vin0, vin1, vpatch, tout, isem0, isem1,
           osem):
    wid = lax.axis_index("s") * NUM_CORES + lax.axis_index("c")
    iota16 = lax.iota(jnp.int32, 16)
    pat = iota16 * EMBED_DIM

    def start_in(c, vbuf, isem):
        v0 = pl.multiple_of(c * VCHUNK, 128)
        pltpu.async_copy(tab_t.at[:, pl.ds(v0, VCHUNK)], vbuf, isem)

    def wait_in(vbuf, isem):
        pltpu.make_async_copy(tab_t.at[:, pl.ds(0, VCHUNK)], vbuf, isem).wait()

    def wait_out():
        pltpu.make_async_copy(tout, tabr.at[pl.ds(0, VCHUNK * EMBED_DIM)],
                              osem).wait()

    def scatter_out(c, vbuf):
        @plsc.parallel_loop(0, EMBED_DIM * (VCHUNK // 16), unroll=8)
        def _(run):
            d = run // (VCHUNK // 16)
            t0 = (run % (VCHUNK // 16)) * 16
            base = jnp.full((16,), t0 * EMBED_DIM + d, jnp.int32)
            plsc.store_scatter(tout, [pat + base], vbuf[d, pl.ds(t0, 16)])
        o0 = pl.multiple_of(c * (VCHUNK * EMBED_DIM), 8)
        pltpu.async_copy(tout, tabr.at[pl.ds(o0, VCHUNK * EMBED_DIM)], osem)

    vins = (vin0, vin1)
    isems = (isem0, isem1)
    c_first = wid
    start_in(c_first, vin0, isem0)

    def pairs(t, carry):
        c0 = wid + NUM_WORKERS * 2 * t
        c1 = c0 + NUM_WORKERS
        c2 = c1 + NUM_WORKERS

        @pl.when(c1 < FULL_CHUNKS)
        def _():
            start_in(c1, vin1, isem1)

        @pl.when(c0 < FULL_CHUNKS)
        def _():
            wait_in(vin0, isem0)
            @pl.when(t > 0)
            def _():
                wait_out()
            scatter_out(c0, vin0)

        @pl.when(c2 < FULL_CHUNKS)
        def _():
            start_in(c2, vin0, isem0)

        @pl.when(c1 < FULL_CHUNKS)
        def _():
            wait_in(vin1, isem1)
            wait_out()
            scatter_out(c1, vin1)
        return carry

    lax.fori_loop(0, 16, pairs, 0)
    wait_out()

    @pl.when(wid == 31)
    def _():
        pltpu.sync_copy(patch_flat, vpatch)
        pltpu.sync_copy(vpatch, tabr.at[pl.ds(TAIL64 * EMBED_DIM, 64 * 32)])

    @pl.when(wid == 30)
    def _():
        pltpu.sync_copy(tab_t.at[:, pl.ds(TAIL512, 512)],
                        vin1.at[:, pl.ds(0, 512)])

        @plsc.parallel_loop(0, EMBED_DIM * (512 // 16), unroll=8)
        def _(run):
            d = run // (512 // 16)
            t0 = (run % (512 // 16)) * 16
            base = jnp.full((16,), t0 * EMBED_DIM + d, jnp.int32)
            plsc.store_scatter(tout, [pat + base], vin1[d, pl.ds(t0, 16)])
        pltpu.sync_copy(tout.at[pl.ds(0, 512 * EMBED_DIM)],
                        tabr.at[pl.ds(TAIL512 * EMBED_DIM, 512 * EMBED_DIM)])


@jax.jit
def _transpose_table(tab_t, patch_flat):
    mesh = plsc.VectorSubcoreMesh(core_axis_name="c", subcore_axis_name="s")
    f = functools.partial(
        pl.kernel,
        mesh=mesh,
        out_type=jax.ShapeDtypeStruct((VOCAB * EMBED_DIM,), jnp.float32),
        scratch_types=[
            pltpu.VMEM((EMBED_DIM, VCHUNK), jnp.float32),
            pltpu.VMEM((EMBED_DIM, VCHUNK), jnp.float32),
            pltpu.VMEM((64 * 32,), jnp.float32),
            pltpu.VMEM((VCHUNK * EMBED_DIM,), jnp.float32),
            pltpu.SemaphoreType.DMA,
            pltpu.SemaphoreType.DMA,
            pltpu.SemaphoreType.DMA,
        ],
        compiler_params=pltpu.CompilerParams(
            use_tc_tiling_on_sc=True, needs_layout_passes=False,
            disable_bounds_checks=True),
    )(_tbody)
    return f(tab_t, patch_flat)


@jax.jit
def _gather(idx_lin, table):
    mesh = plsc.VectorSubcoreMesh(core_axis_name="c", subcore_axis_name="s")
    f = functools.partial(
        pl.kernel,
        mesh=mesh,
        out_type=jax.ShapeDtypeStruct(
            (SEQ, SUB_TILES, (BATCH // 128) * 1024), jnp.float32),
        scratch_types=[
            pltpu.VMEM((JOBS_PER_W, CHUNK), jnp.int32),
            pltpu.VMEM((CHUNK, EMBED_DIM), jnp.float32),
            pltpu.VMEM((CHUNK, EMBED_DIM), jnp.float32),
            pltpu.VMEM((T5,), jnp.float32),
            pltpu.VMEM((T5,), jnp.float32),
            pltpu.SemaphoreType.DMA,
            pltpu.SemaphoreType.DMA,
            pltpu.SemaphoreType.DMA,
            pltpu.SemaphoreType.DMA,
        ],
        compiler_params=pltpu.CompilerParams(
            use_tc_tiling_on_sc=False, needs_layout_passes=False,
            disable_bounds_checks=True),
    )(_body)
    return f(idx_lin, table)


def kernel(sentence_tokens, embedding_table):
    idx_lin = sentence_tokens.T.reshape(NUM_JOBS, CHUNK).astype(jnp.int32)
    patch_flat = embedding_table[TAIL64:, :].reshape(-1)
    tabr = _transpose_table(embedding_table.T, patch_flat)
    out3 = _gather(idx_lin, tabr.reshape(VOCAB, EMBED_DIM))
    out5 = out3.reshape(SEQ, SUB_TILES, BATCH // 128, 8, 128)
    res = out5.transpose(0, 1, 3, 2, 4).reshape(SEQ, EMBED_DIM, BATCH)
    return res.transpose(2, 0, 1)


# speedup vs baseline: 1.2359x; 1.1711x over previous
import functools

import jax
import jax.numpy as jnp
from jax import lax
from jax.experimental import pallas as pl
from jax.experimental.pallas import tpu as pltpu
from jax.experimental.pallas import tpu_sc as plsc

VOCAB = 1000000
EMBED_DIM = 32
BATCH = 4096
SEQ = 200

NUM_CORES = 2
NUM_SUBCORES = 16
NUM_WORKERS = NUM_CORES * NUM_SUBCORES

N = BATCH * SEQ
CHUNK = 512
JOBS_PER_SEQ = BATCH // CHUNK
NUM_JOBS = SEQ * JOBS_PER_SEQ
JOBS_PER_W = NUM_JOBS // NUM_WORKERS
PAIRS = JOBS_PER_W // 2
LANE_TILES = CHUNK // 128
SUB_TILES = EMBED_DIM // 8
T5 = CHUNK * EMBED_DIM
RUN = T5 // SUB_TILES


def _body(idx_hbm, table_hbm, out_hbm,
          idx_all, rows0, rows1, t50, t51, gsem0, gsem1, ssem0, ssem1):
    wid = lax.axis_index("s") * NUM_CORES + lax.axis_index("c")
    job_base = wid * JOBS_PER_W
    iota16 = lax.iota(jnp.int32, 16)
    p0 = (iota16 // 8) * 4096 + (iota16 % 8) * 128
    p1 = p0 + 2 * 4096

    pltpu.sync_copy(idx_hbm.at[pl.ds(wid * JOBS_PER_W, JOBS_PER_W)], idx_all)

    def store_job(t5f, job_id, sem):
        s = job_id // JOBS_PER_SEQ
        c0 = (job_id % JOBS_PER_SEQ) * LANE_TILES * 1024
        for i in range(SUB_TILES):
            pltpu.async_copy(t5f.at[pl.ds(i * RUN, RUN)],
                             out_hbm.at[s, i, pl.ds(c0, RUN)], sem)

    def drain_store(t5f, sem):
        for i in range(SUB_TILES):
            pltpu.make_async_copy(t5f.at[pl.ds(i * RUN, RUN)],
                                  out_hbm.at[0, i, pl.ds(0, RUN)], sem).wait()

    def transpose(rows, t5f):
        @plsc.parallel_loop(0, CHUNK, unroll=8)
        def _(b):
            base = (b // 128) * 1024 + (b % 128)
            bb = jnp.full((16,), base, jnp.int32)
            plsc.store_scatter(t5f, [p0 + bb], rows[b, 0:16])
            plsc.store_scatter(t5f, [p1 + bb], rows[b, 16:32])

    pltpu.async_copy(table_hbm.at[idx_all.at[0]], rows0, gsem0)

    def pair(t, carry):
        ja = 2 * t
        jb = 2 * t + 1

        gb = pltpu.async_copy(table_hbm.at[idx_all.at[jb]], rows1, gsem1)

        pltpu.make_async_copy(table_hbm.at[idx_all.at[ja]], rows0, gsem0).wait()
        @pl.when(t > 0)
        def _():
            drain_store(t50, ssem0)
        transpose(rows0, t50)
        store_job(t50, job_base + ja, ssem0)

        nxt = jnp.minimum(2 * t + 2, JOBS_PER_W - 2)
        pltpu.async_copy(table_hbm.at[idx_all.at[nxt]], rows0, gsem0)

        gb.wait()
        @pl.when(t > 0)
        def _():
            drain_store(t51, ssem1)
        transpose(rows1, t51)
        store_job(t51, job_base + jb, ssem1)
        return carry

    lax.fori_loop(0, PAIRS, pair, 0)

    pltpu.make_async_copy(table_hbm.at[idx_all.at[JOBS_PER_W - 2]],
                          rows0, gsem0).wait()
    drain_store(t50, ssem0)
    drain_store(t51, ssem1)


@jax.jit
def _gather(idx_lin, table):
    mesh = plsc.VectorSubcoreMesh(core_axis_name="c", subcore_axis_name="s")
    f = functools.partial(
        pl.kernel,
        mesh=mesh,
        out_type=jax.ShapeDtypeStruct(
            (SEQ, SUB_TILES, (BATCH // 128) * 1024), jnp.float32),
        scratch_types=[
            pltpu.VMEM((JOBS_PER_W, CHUNK), jnp.int32),
            pltpu.VMEM((CHUNK, EMBED_DIM), jnp.float32),
            pltpu.VMEM((CHUNK, EMBED_DIM), jnp.float32),
            pltpu.VMEM((T5,), jnp.float32),
            pltpu.VMEM((T5,), jnp.float32),
            pltpu.SemaphoreType.DMA,
            pltpu.SemaphoreType.DMA,
            pltpu.SemaphoreType.DMA,
            pltpu.SemaphoreType.DMA,
        ],
        compiler_params=pltpu.CompilerParams(
            use_tc_tiling_on_sc=False, needs_layout_passes=False,
            disable_bounds_checks=True),
    )(_body)
    return f(idx_lin, table)


def kernel(sentence_tokens, embedding_table):
    idx_lin = sentence_tokens.T.reshape(NUM_JOBS, CHUNK).astype(jnp.int32)
    out3 = _gather(idx_lin, embedding_table)
    out5 = out3.reshape(SEQ, SUB_TILES, BATCH // 128, 8, 128)
    res = out5.transpose(0, 1, 3, 2, 4).reshape(SEQ, EMBED_DIM, BATCH)
    return res.transpose(2, 0, 1)
